# baseline (speedup 1.0000x reference)
import functools

import jax
import jax.numpy as jnp
from jax import lax
from jax.experimental import pallas as pl
from jax.experimental.pallas import tpu as pltpu
from jax.experimental.pallas import tpu_sc as plsc

N_E = 8192
E_DIM = 256
BETA = 0.25
B_TOT = 16 * 576
B_HALF = B_TOT // 2

TM = 512
TN = 512
HALF_TILES = B_HALF // TM
N_COL_CHUNKS = N_E // TN


def _argmin_body(rn_ref, en_ref, z_ref, e_ref, idx_ref, loss_ref):
    i = pl.program_id(0)
    z_tile = z_ref[...]
    rn_tile = rn_ref[...]

    STRIPS = 4
    SW = TN // STRIPS
    m_st = [None] * STRIPS
    c_st = [None] * STRIPS
    rn_col = rn_tile[:, None]
    mms = [
        lax.dot_general(z_tile, e_ref[pl.ds(p * 4 * TN, 4 * TN), :],
                        (((1,), (1,)), ((), ())),
                        preferred_element_type=jnp.float32)
        for p in range(4)
    ]

    def d_of(c, s):
        o = c * TN + s * SW
        lo = (c % 4) * TN + s * SW
        return (rn_col + en_ref[pl.ds(o, SW)][None, :]) \
            - mms[c // 4][:, lo:lo + SW]

    def tour(chunks, s):
        if len(chunks) == 2:
            dl, dr = d_of(chunks[0], s), d_of(chunks[1], s)
            u = dr < dl
            return (jnp.where(u, dr, dl),
                    jnp.where(u, jnp.int32(chunks[1]), jnp.int32(chunks[0])))
        h = len(chunks) // 2
        lv, lc = tour(chunks[:h], s)
        rv, rc = tour(chunks[h:], s)
        u = rv < lv
        return jnp.where(u, rv, lv), jnp.where(u, rc, lc)

    for s in range(STRIPS):
        m_st[s], c_st[s] = tour(list(range(N_COL_CHUNKS)), s)

    ms = list(m_st)
    while len(ms) > 1:
        ms = [jnp.minimum(ms[k], ms[k + 1]) for k in range(0, len(ms), 2)]
    m = jnp.min(ms[0], axis=1)
    siota = lax.broadcasted_iota(jnp.int32, (TM, SW), 1)
    BIG = jnp.int32(2**30)
    cand = jnp.full((TM, SW), BIG, jnp.int32)
    for s in range(STRIPS):
        gidx = c_st[s] * TN + (siota + jnp.int32(s * SW))
        cand = jnp.minimum(cand,
                           jnp.where(m_st[s] == m[:, None], gidx, BIG))
    ix = jnp.min(cand, axis=1)

    idx_ref[0, 0, :] = ix
    part = jnp.sum(m)

    @pl.when(i == 0)
    def _():
        loss_ref[0, 0] = part

    @pl.when(i > 0)
    def _():
        loss_ref[0, 0] += part

    @pl.when(i == HALF_TILES - 1)
    def _():
        loss_ref[0, 0] = loss_ref[0, 0] * ((1.0 + BETA) / (B_TOT * E_DIM))


def _make_argmin_call(tile_off):
    return pl.pallas_call(
        _argmin_body,
        grid=(HALF_TILES,),
        in_specs=[
            pl.BlockSpec((TM,), lambda i: (i + tile_off,)),
            pl.BlockSpec((N_E,), lambda i: (0,)),
            pl.BlockSpec((TM, E_DIM), lambda i: (i + tile_off, 0)),
            pl.BlockSpec((N_E, E_DIM), lambda i: (0, 0)),
        ],
        out_specs=[
            pl.BlockSpec((1, 1, TM), lambda i: (i, 0, 0)),
            pl.BlockSpec(memory_space=pltpu.SMEM),
        ],
        out_shape=[
            jax.ShapeDtypeStruct((HALF_TILES, 1, TM), jnp.int32),
            jax.ShapeDtypeStruct((1, 1), jnp.float32),
        ],
    )


_argmin_a = _make_argmin_call(0)
_argmin_b = _make_argmin_call(HALF_TILES)

_NW = 32
_RPW = B_HALF // _NW
_CH = 2
_CW = _RPW // _CH
_HSL = N_E // 16

_sc_mesh = plsc.VectorSubcoreMesh(core_axis_name="c", subcore_axis_name="s")


@functools.partial(
    pl.kernel,
    out_type=[
        jax.ShapeDtypeStruct((B_HALF, E_DIM), jnp.float32),
        jax.ShapeDtypeStruct((2, N_E), jnp.float32),
    ],
    mesh=_sc_mesh,
    scratch_types=[
        pltpu.VMEM((_CH, _CW), jnp.int32),
        pltpu.VMEM((_RPW, E_DIM), jnp.float32),
        pltpu.VMEM((_CW,), jnp.float32),
        pltpu.VMEM((_HSL,), jnp.float32),
        pltpu.VMEM_SHARED((N_E,), jnp.float32),
        pltpu.SemaphoreType.DMA,
    ],
)
def _sc_gather_hist(emb_hbm, idx_hbm, zq_hbm, cnt_hbm,
                    idx_v, rows_v, ones_v, zer_v, hist_sh, sem):
    c = lax.axis_index("c")
    s = lax.axis_index("s")
    wid = s * 2 + c
    pltpu.sync_copy(idx_hbm.at[wid], idx_v)

    handles = [
        pltpu.async_copy(emb_hbm.at[idx_v.at[j]],
                         rows_v.at[pl.ds(j * _CW, _CW)], sem)
        for j in range(_CH)
    ]

    for k in range(_HSL // 16):
        zer_v[pl.ds(k * 16, 16)] = jnp.zeros((16,), jnp.float32)
    for k in range(_CW // 16):
        ones_v[pl.ds(k * 16, 16)] = jnp.ones((16,), jnp.float32)
    pltpu.sync_copy(zer_v, hist_sh.at[pl.ds(s * _HSL, _HSL)])
    plsc.subcore_barrier()

    for j in range(_CH):
        pltpu.sync_copy(ones_v, hist_sh.at[idx_v.at[j]], add=True)
    plsc.subcore_barrier()

    @pl.when(s == 0)
    def _():
        pltpu.sync_copy(hist_sh, cnt_hbm.at[c])

    for h in handles:
        h.wait()
    pltpu.sync_copy(rows_v, zq_hbm.at[pl.ds(wid * _RPW, _RPW)])


def _perp_body(cnt_ref, out_ref):
    cnt = cnt_ref[...]
    tot = (cnt[0:8, :] + cnt[8:16, :]) + (cnt[16:24, :] + cnt[24:32, :])
    avg = tot / jnp.float32(B_TOT)
    ent = jnp.sum(avg * jnp.log(avg + 1e-12))
    out_ref[0, 0] = jnp.exp(-ent)


_perp_call = pl.pallas_call(
    _perp_body,
    in_specs=[pl.BlockSpec((32, 1024), lambda: (0, 0))],
    out_specs=pl.BlockSpec(memory_space=pltpu.SMEM),
    out_shape=jax.ShapeDtypeStruct((1, 1), jnp.float32),
)


def kernel(z, embedding):
    bz = z.shape[0]
    z_flat = z.reshape(-1, E_DIM)
    rn = jnp.sum(z_flat ** 2, axis=1)
    en = jnp.sum(embedding ** 2, axis=1)
    e2 = embedding * 2.0

    idx_a, loss_a = _argmin_a(rn, en, z_flat, e2)
    zq_a, cnt_a = _sc_gather_hist(embedding,
                                  idx_a.reshape(_NW, _CH, _CW))
    idx_b, loss_b = _argmin_b(rn, en, z_flat, e2)
    zq_b, cnt_b = _sc_gather_hist(embedding,
                                  idx_b.reshape(_NW, _CH, _CW))

    idx_flat = jnp.concatenate([idx_a.reshape(B_HALF), idx_b.reshape(B_HALF)])
    counts = jnp.concatenate([cnt_a, cnt_b], axis=0)
    perp = _perp_call(counts.reshape(32, 1024))
    loss = loss_a.reshape(()) + loss_b.reshape(())

    z_q = jnp.concatenate([zq_a, zq_b], axis=0).reshape(bz, -1, E_DIM)
    return (z_q, loss, idx_flat, perp.reshape(()))


# speedup vs baseline: 1.0870x; 1.0870x over previous
import functools

import jax
import jax.numpy as jnp
from jax import lax
from jax.experimental import pallas as pl
from jax.experimental.pallas import tpu as pltpu
from jax.experimental.pallas import tpu_sc as plsc

N_E = 8192
E_DIM = 256
BETA = 0.25
B_TOT = 16 * 576

TM = 512
TN = 512
N_ROW_TILES = B_TOT // TM
N_COL_CHUNKS = N_E // TN


def _argmin_body(rn_ref, en_ref, z_ref, e_ref, idx_ref, loss_ref):
    i = pl.program_id(0)
    z_tile = z_ref[...]
    rn_tile = rn_ref[...]

    STRIPS = 4
    SW = TN // STRIPS
    m_st = [None] * STRIPS
    c_st = [None] * STRIPS
    rn_col = rn_tile[:, None]
    mms = [
        lax.dot_general(z_tile, e_ref[pl.ds(p * 4 * TN, 4 * TN), :],
                        (((1,), (1,)), ((), ())),
                        preferred_element_type=jnp.float32)
        for p in range(4)
    ]
    def d_of(c, s):
        o = c * TN + s * SW
        lo = (c % 4) * TN + s * SW
        return (rn_col + en_ref[pl.ds(o, SW)][None, :]) \
            - mms[c // 4][:, lo:lo + SW]

    def tour(chunks, s):
        if len(chunks) == 2:
            dl, dr = d_of(chunks[0], s), d_of(chunks[1], s)
            u = dr < dl
            return (jnp.where(u, dr, dl),
                    jnp.where(u, jnp.int32(chunks[1]), jnp.int32(chunks[0])))
        h = len(chunks) // 2
        lv, lc = tour(chunks[:h], s)
        rv, rc = tour(chunks[h:], s)
        u = rv < lv
        return jnp.where(u, rv, lv), jnp.where(u, rc, lc)

    for s in range(STRIPS):
        m_st[s], c_st[s] = tour(list(range(N_COL_CHUNKS)), s)

    ms = list(m_st)
    while len(ms) > 1:
        ms = [jnp.minimum(ms[k], ms[k + 1]) for k in range(0, len(ms), 2)]
    mrow = ms[0]
    m = jnp.min(mrow, axis=1)
    siota = lax.broadcasted_iota(jnp.int32, (TM, SW), 1)
    BIG = jnp.int32(2**30)
    cand = jnp.full((TM, SW), BIG, jnp.int32)
    for s in range(STRIPS):
        gidx = c_st[s] * TN + (siota + jnp.int32(s * SW))
        cand = jnp.minimum(cand,
                           jnp.where(m_st[s] == m[:, None], gidx, BIG))
    ix = jnp.min(cand, axis=1)

    idx_ref[0, 0, :] = ix
    part = jnp.sum(m)

    @pl.when(i == 0)
    def _():
        loss_ref[0, 0] = part

    @pl.when(i > 0)
    def _():
        loss_ref[0, 0] += part

    @pl.when(i == N_ROW_TILES - 1)
    def _():
        loss_ref[0, 0] = loss_ref[0, 0] * ((1.0 + BETA) / (B_TOT * E_DIM))


_argmin_call = pl.pallas_call(
    _argmin_body,
    grid=(N_ROW_TILES,),
    in_specs=[
        pl.BlockSpec((TM,), lambda i: (i,)),
        pl.BlockSpec((N_E,), lambda i: (0,)),
        pl.BlockSpec((TM, E_DIM), lambda i: (i, 0)),
        pl.BlockSpec((N_E, E_DIM), lambda i: (0, 0)),
    ],
    out_specs=[
        pl.BlockSpec((1, 1, TM), lambda i: (i, 0, 0)),
        pl.BlockSpec(memory_space=pltpu.SMEM),
    ],
    out_shape=[
        jax.ShapeDtypeStruct((N_ROW_TILES, 1, TM), jnp.int32),
        jax.ShapeDtypeStruct((1, 1), jnp.float32),
    ],
)

_NW = 32
_RPW = B_TOT // _NW
_CH = 3
_CW = _RPW // _CH
_HSL = N_E // 16

_sc_mesh = plsc.VectorSubcoreMesh(core_axis_name="c", subcore_axis_name="s")


@functools.partial(
    pl.kernel,
    out_type=[
        jax.ShapeDtypeStruct((B_TOT, E_DIM), jnp.float32),
        jax.ShapeDtypeStruct((2, N_E), jnp.float32),
    ],
    mesh=_sc_mesh,
    scratch_types=[
        pltpu.VMEM((_CH, _CW), jnp.int32),
        pltpu.VMEM((_RPW, E_DIM), jnp.float32),
        pltpu.VMEM((_CW,), jnp.float32),
        pltpu.VMEM((_HSL,), jnp.float32),
        pltpu.VMEM_SHARED((N_E,), jnp.float32),
        pltpu.SemaphoreType.DMA,
    ],
)
def _sc_gather_hist(emb_hbm, idx_hbm, zq_hbm, cnt_hbm,
                    idx_v, rows_v, ones_v, zer_v, hist_sh, sem):
    c = lax.axis_index("c")
    s = lax.axis_index("s")
    wid = s * 2 + c
    pltpu.sync_copy(idx_hbm.at[wid], idx_v)

    handles = [
        pltpu.async_copy(emb_hbm.at[idx_v.at[j]],
                         rows_v.at[pl.ds(j * _CW, _CW)], sem)
        for j in range(_CH)
    ]

    for k in range(_HSL // 16):
        zer_v[pl.ds(k * 16, 16)] = jnp.zeros((16,), jnp.float32)
    for k in range(_CW // 16):
        ones_v[pl.ds(k * 16, 16)] = jnp.ones((16,), jnp.float32)
    pltpu.sync_copy(zer_v, hist_sh.at[pl.ds(s * _HSL, _HSL)])
    plsc.subcore_barrier()

    for j in range(_CH):
        pltpu.sync_copy(ones_v, hist_sh.at[idx_v.at[j]], add=True)
    plsc.subcore_barrier()

    @pl.when(s == 0)
    def _():
        pltpu.sync_copy(hist_sh, cnt_hbm.at[c])

    for h in handles:
        h.wait()
    pltpu.sync_copy(rows_v, zq_hbm.at[pl.ds(wid * _RPW, _RPW)])


def _perp_body(cnt_ref, out_ref):
    cnt = cnt_ref[...]
    tot = cnt[0:8, :] + cnt[8:16, :]
    avg = tot / jnp.float32(B_TOT)
    ent = jnp.sum(avg * jnp.log(avg + 1e-12))
    out_ref[0, 0] = jnp.exp(-ent)


_perp_call = pl.pallas_call(
    _perp_body,
    in_specs=[pl.BlockSpec((16, 1024), lambda: (0, 0))],
    out_specs=pl.BlockSpec(memory_space=pltpu.SMEM),
    out_shape=jax.ShapeDtypeStruct((1, 1), jnp.float32),
)


def kernel(z, embedding):
    bz = z.shape[0]
    z_flat = z.reshape(-1, E_DIM)
    rn = jnp.sum(z_flat ** 2, axis=1)
    en = jnp.sum(embedding ** 2, axis=1)

    idx_tiles, loss = _argmin_call(rn, en, z_flat, embedding * 2.0)
    idx_flat = idx_tiles.reshape(B_TOT)

    zq_flat, counts = _sc_gather_hist(embedding,
                                      idx_flat.reshape(_NW, _CH, _CW))
    perp = _perp_call(counts.reshape(16, 1024))

    z_q = zq_flat.reshape(bz, -1, E_DIM)
    return (z_q, loss.reshape(()), idx_flat, perp.reshape(()))
